# baseline (speedup 1.0000x reference)
import functools

import jax
import jax.numpy as jnp
from jax import lax
from jax.experimental import pallas as pl
from jax.experimental.pallas import tpu as pltpu
from jax.experimental.pallas import tpu_sc as plsc

NC = 2
NS = 16
K = 120
NB = 3
BLK = 6
ZU = 100


def _fill(ref, nrows, ncols, val):
    v = jnp.full((16,), val, jnp.float32)
    npieces = ncols // 16

    def body(k, _):
        i = k // npieces
        j = k % npieces
        ref[i, pl.ds(j * 16, 16)] = v
        return 0

    lax.fori_loop(0, nrows * npieces, body, 0)


def _make_sc_agg(N, EP, D):
    n = EP // (K * NC * NS)
    nblk = n // BLK
    units = N // ZU
    rounds = (units + NS - 1) // NS
    assert n * K * NC * NS == EP and nblk * BLK == n and units * ZU == N
    assert BLK == 2 * NB and K >= ZU

    scratch = [
        pltpu.VMEM((2, BLK, K), jnp.int32),
        pltpu.VMEM((2, BLK, K), jnp.int32),
        pltpu.VMEM_SHARED((N + 8, D), jnp.float32),
    ]
    scratch += [pltpu.VMEM((K, D), jnp.float32) for _ in range(NB)]
    scratch += [pltpu.SemaphoreType.DMA for _ in range(2 * NB + 1)]

    mesh = plsc.VectorSubcoreMesh(core_axis_name="c", subcore_axis_name="s")

    @functools.partial(
        pl.kernel,
        mesh=mesh,
        out_type=jax.ShapeDtypeStruct((NC * N, D), jnp.float32),
        scratch_types=scratch,
        compiler_params=pltpu.CompilerParams(use_tc_tiling_on_sc=False),
    )
    def k(x_hbm, ei_hbm, agg_out, idx_s, idx_d, agg_sh, *bufsem):
        bufs = bufsem[:NB]
        gsem = bufsem[NB:2 * NB]
        ssem = bufsem[2 * NB:3 * NB]
        isem = bufsem[3 * NB]
        cid = lax.axis_index("c")
        sid = lax.axis_index("s")

        _fill(bufs[0], K, D, 0.0)
        zsrc = bufs[0].at[pl.ds(0, ZU)]
        for r in range(rounds):
            u = r * NS + sid

            @pl.when(u < units)
            def _():
                pltpu.sync_copy(zsrc, agg_sh.at[pl.ds(u * ZU, ZU)])

        plsc.subcore_barrier()

        base = (cid * NS + sid) * n

        def i_start(t):
            pltpu.async_copy(
                ei_hbm.at[0, pl.ds(base + t * BLK, BLK)], idx_s.at[t % 2], isem
            )
            pltpu.async_copy(
                ei_hbm.at[1, pl.ds(base + t * BLK, BLK)], idx_d.at[t % 2], isem
            )

        def i_wait(t):
            pltpu.make_async_copy(
                ei_hbm.at[0, pl.ds(base + t * BLK, BLK)], idx_s.at[t % 2], isem
            ).wait()
            pltpu.make_async_copy(
                ei_hbm.at[1, pl.ds(base + t * BLK, BLK)], idx_d.at[t % 2], isem
            ).wait()

        def srow(c, j):
            return idx_s.at[(c // BLK) % 2, j]

        def drow(c, j):
            return idx_d.at[(c // BLK) % 2, j]

        def g_start(c, j, b):
            pltpu.async_copy(x_hbm.at[srow(c, j)], bufs[b], gsem[b])

        def g_wait(c, j, b):
            pltpu.make_async_copy(x_hbm.at[srow(c, j)], bufs[b], gsem[b]).wait()

        def s_start(c, j, b):
            pltpu.async_copy(bufs[b], agg_sh.at[drow(c, j)], ssem[b], add=True)

        def s_wait(c, j, b):
            pltpu.make_async_copy(bufs[b], agg_sh.at[drow(c, j)], ssem[b]).wait()

        i_start(0)
        i_wait(0)
        i_start(1)
        for b in range(NB - 1):
            g_start(b, b, b)

        def body(u, _):
            for j in range(BLK):
                c = u * BLK + j
                b = j % NB
                g_wait(c, j, b)
                s_start(c, j, b)
                if j == 0:
                    @pl.when(u >= 1)
                    def _():
                        s_wait(c - 1, BLK - 1, (BLK - 1) % NB)

                    @pl.when((u >= 1) & (u < nblk - 1))
                    def _():
                        i_start(u + 1)
                else:
                    s_wait(c - 1, j - 1, (j - 1) % NB)
                if j == BLK - 2:
                    @pl.when(u < nblk - 1)
                    def _():
                        i_wait(u + 1)
                pj = j + NB - 1
                if pj < BLK:
                    g_start(c + NB - 1, pj, (pj) % NB)
                else:
                    @pl.when(u < nblk - 1)
                    def _():
                        g_start(c + NB - 1, pj - BLK, (pj - BLK) % NB)

            return 0

        lax.fori_loop(0, nblk, body, 0)
        s_wait(n - 1, BLK - 1, (BLK - 1) % NB)
        plsc.subcore_barrier()

        for r in range(rounds):
            u = r * NS + sid

            @pl.when(u < units)
            def _():
                pltpu.sync_copy(
                    agg_sh.at[pl.ds(u * ZU, ZU)],
                    agg_out.at[pl.ds(cid * N + u * ZU, ZU)],
                )

    return k


def _make_sc_deg(N, EP, deg_w=16):
    n = EP // (K * NC * NS)
    units = N // ZU
    rounds = (units + NS - 1) // NS
    batch = BLK
    assert n * K * NC * NS == EP and units * ZU == N and n % batch == 0

    mesh = plsc.VectorSubcoreMesh(core_axis_name="c", subcore_axis_name="s")

    @functools.partial(
        pl.kernel,
        mesh=mesh,
        out_type=jax.ShapeDtypeStruct((NC * N, deg_w), jnp.float32),
        scratch_types=[
            pltpu.VMEM((n, K), jnp.int32),
            pltpu.VMEM((K, deg_w), jnp.float32),
            pltpu.VMEM_SHARED((N + 8, deg_w), jnp.float32),
            pltpu.SemaphoreType.DMA,
        ],
        compiler_params=pltpu.CompilerParams(use_tc_tiling_on_sc=False),
    )
    def k(ei_hbm, deg_out, idx_d, ones_v, deg_sh, sem):
        cid = lax.axis_index("c")
        sid = lax.axis_index("s")

        _fill(ones_v, K, deg_w, 0.0)
        zsrc = ones_v.at[pl.ds(0, ZU)]
        for r in range(rounds):
            u = r * NS + sid

            @pl.when(u < units)
            def _():
                pltpu.sync_copy(zsrc, deg_sh.at[pl.ds(u * ZU, ZU)])

        _fill(ones_v, K, deg_w, 1.0)
        plsc.subcore_barrier()

        base = (cid * NS + sid) * n
        pltpu.sync_copy(ei_hbm.at[1, pl.ds(base, n)], idx_d)

        def body(g, _):
            for j in range(batch):
                c = g * batch + j
                pltpu.async_copy(ones_v, deg_sh.at[idx_d.at[c]], sem, add=True)
            for j in range(batch):
                c = g * batch + j
                pltpu.make_async_copy(ones_v, deg_sh.at[idx_d.at[c]], sem).wait()
            return 0

        lax.fori_loop(0, n // batch, body, 0)
        plsc.subcore_barrier()

        for r in range(rounds):
            u = r * NS + sid

            @pl.when(u < units)
            def _():
                pltpu.sync_copy(
                    deg_sh.at[pl.ds(u * ZU, ZU)],
                    deg_out.at[pl.ds(cid * N + u * ZU, ZU)],
                )

    return k


def _tc_layer(N, D, H, bn, deg_w, relu, w2_cols=None):
    grid = (N // bn,)

    def body(x_ref, agg_ref, agg2_ref, deg_ref, deg2_ref, ws_ref, wn_ref, b_ref,
             *rest):
        deg = (deg_ref[...] + deg2_ref[...])[:, :1]
        mean = (agg_ref[...] + agg2_ref[...]) / jnp.maximum(deg, 1.0)
        h = (
            jnp.dot(x_ref[...], ws_ref[...], preferred_element_type=jnp.float32)
            + jnp.dot(mean, wn_ref[...], preferred_element_type=jnp.float32)
            + b_ref[...]
        )
        if relu:
            h = jnp.maximum(h, 0.0)
        if w2_cols is not None:
            w2_ref, o_ref, y_ref = rest
            o_ref[...] = h
            y_ref[...] = jnp.dot(h, w2_ref[...], preferred_element_type=jnp.float32)
        else:
            (o_ref,) = rest
            o_ref[...] = h

    in_specs = [
        pl.BlockSpec((bn, D), lambda i: (i, 0)),
        pl.BlockSpec((bn, H), lambda i: (i, 0)),
        pl.BlockSpec((bn, H), lambda i: (i + N // bn, 0)),
        pl.BlockSpec((bn, deg_w), lambda i: (i, 0)),
        pl.BlockSpec((bn, deg_w), lambda i: (i + N // bn, 0)),
        pl.BlockSpec((D, H), lambda i: (0, 0)),
        pl.BlockSpec((H, H), lambda i: (0, 0)),
        pl.BlockSpec((1, H), lambda i: (0, 0)),
    ]
    out_shape = [jax.ShapeDtypeStruct((N, H), jnp.float32)]
    out_specs = [pl.BlockSpec((bn, H), lambda i: (i, 0))]
    if w2_cols is not None:
        in_specs.append(pl.BlockSpec((H, w2_cols), lambda i: (0, 0)))
        out_shape.append(jax.ShapeDtypeStruct((N, w2_cols), jnp.float32))
        out_specs.append(pl.BlockSpec((bn, w2_cols), lambda i: (i, 0)))

    return pl.pallas_call(
        body,
        grid=grid,
        in_specs=in_specs,
        out_specs=out_specs if w2_cols is not None else out_specs[0],
        out_shape=out_shape if w2_cols is not None else out_shape[0],
    )


def _tc_layer3(N, D, CP, C, bn, deg_w):
    grid = (N // bn,)

    def body(x_ref, agg_ref, agg2_ref, deg_ref, deg2_ref, ws_ref, b_ref, o_ref):
        deg = (deg_ref[...] + deg2_ref[...])[:, :1]
        mean = (agg_ref[...] + agg2_ref[...]) / jnp.maximum(deg, 1.0)
        h = (
            jnp.dot(x_ref[...], ws_ref[...], preferred_element_type=jnp.float32)
            + mean
            + b_ref[...]
        )
        col = lax.broadcasted_iota(jnp.int32, h.shape, 1)
        hm = jnp.where(col < C, h, -1e30)
        m = jnp.max(hm, axis=-1, keepdims=True)
        e = jnp.where(col < C, jnp.exp(hm - m), 0.0)
        s = jnp.sum(e, axis=-1, keepdims=True)
        o_ref[...] = (hm - m - jnp.log(s))[:, :C]

    return pl.pallas_call(
        body,
        grid=grid,
        in_specs=[
            pl.BlockSpec((bn, D), lambda i: (i, 0)),
            pl.BlockSpec((bn, CP), lambda i: (i, 0)),
            pl.BlockSpec((bn, CP), lambda i: (i + N // bn, 0)),
            pl.BlockSpec((bn, deg_w), lambda i: (i, 0)),
            pl.BlockSpec((bn, deg_w), lambda i: (i + N // bn, 0)),
            pl.BlockSpec((D, CP), lambda i: (0, 0)),
            pl.BlockSpec((1, CP), lambda i: (0, 0)),
        ],
        out_specs=pl.BlockSpec((bn, C), lambda i: (i, 0)),
        out_shape=jax.ShapeDtypeStruct((N, C), jnp.float32),
    )


def kernel(nfeat, edge_index, W_self1, W_neigh1, b1, W_self2, W_neigh2, b2,
           W_self3, W_neigh3, b3):
    N, D = nfeat.shape
    E = edge_index.shape[1]
    H = W_self1.shape[1]
    C = W_self3.shape[1]
    CP = 48
    deg_w = 16
    bn = 2000

    quant = K * NC * NS * BLK
    EP = ((E + quant - 1) // quant) * quant
    pad = jnp.stack([
        jnp.zeros((EP - E,), jnp.int32),
        jnp.full((EP - E,), N, jnp.int32),
    ])
    ei3 = jnp.concatenate([edge_index, pad], axis=1).reshape(2, EP // K, K)

    Wn3p = jnp.pad(W_neigh3, ((0, 0), (0, CP - C)))
    Ws3p = jnp.pad(W_self3, ((0, 0), (0, CP - C)))
    b3p = jnp.pad(b3, (0, CP - C)).reshape(1, CP)

    deg = _make_sc_deg(N, EP, deg_w)(ei3)
    agg1 = _make_sc_agg(N, EP, D)(nfeat, ei3)
    h1 = _tc_layer(N, D, H, bn, deg_w, True)(
        nfeat, agg1, agg1, deg, deg, W_self1, W_neigh1, b1.reshape(1, H)
    )
    agg2 = _make_sc_agg(N, EP, H)(h1, ei3)
    h2, y3 = _tc_layer(N, H, H, bn, deg_w, True, w2_cols=CP)(
        h1, agg2, agg2, deg, deg, W_self2, W_neigh2, b2.reshape(1, H), Wn3p
    )
    agg3 = _make_sc_agg(N, EP, CP)(y3, ei3)
    return _tc_layer3(N, H, CP, C, bn, deg_w)(h2, agg3, agg3, deg, deg, Ws3p, b3p)


# speedup vs baseline: 1.3610x; 1.3610x over previous
import functools

import jax
import jax.numpy as jnp
from jax import lax
from jax.experimental import pallas as pl
from jax.experimental.pallas import tpu as pltpu
from jax.experimental.pallas import tpu_sc as plsc

NC = 2
NS = 16
K = 50
NB = 4
KD = 100


def _fill(ref, nrows, ncols, val):
    v = jnp.full((16,), val, jnp.float32)
    npieces = ncols // 16

    def body(k, _):
        i = k // npieces
        j = k % npieces
        ref[i, pl.ds(j * 16, 16)] = v
        return 0

    lax.fori_loop(0, nrows * npieces, body, 0)


def _make_sc_agg(N, E, D):
    n_chunk_rows = E // K
    n = n_chunk_rows // (NC * NS)
    units = N // K
    rounds = (units + NS - 1) // NS
    assert n_chunk_rows * K == E and n * NC * NS == n_chunk_rows
    assert units * K == N
    assert n % NB == 0 and n >= 2 * NB

    scratch = [
        pltpu.VMEM((n, K), jnp.int32),
        pltpu.VMEM((n, K), jnp.int32),
        pltpu.VMEM_SHARED((N, D), jnp.float32),
    ]
    scratch += [pltpu.VMEM((K, D), jnp.float32) for _ in range(NB)]
    scratch += [pltpu.SemaphoreType.DMA for _ in range(2 * NB)]

    mesh = plsc.VectorSubcoreMesh(core_axis_name="c", subcore_axis_name="s")

    @functools.partial(
        pl.kernel,
        mesh=mesh,
        out_type=jax.ShapeDtypeStruct((NC * N, D), jnp.float32),
        scratch_types=scratch,
        compiler_params=pltpu.CompilerParams(use_tc_tiling_on_sc=False),
    )
    def k(x_hbm, ei_hbm, agg_out, idx_s, idx_d, agg_sh, *bufsem):
        bufs = bufsem[:NB]
        gsem = bufsem[NB:2 * NB]
        ssem = bufsem[2 * NB:]
        cid = lax.axis_index("c")
        sid = lax.axis_index("s")

        _fill(bufs[0], K, D, 0.0)
        for r in range(rounds):
            u = r * NS + sid

            @pl.when(u < units)
            def _():
                pltpu.sync_copy(bufs[0], agg_sh.at[pl.ds(u * K, K)])

        plsc.subcore_barrier()

        base = (cid * NS + sid) * n
        pltpu.sync_copy(ei_hbm.at[0, pl.ds(base, n)], idx_s)
        pltpu.sync_copy(ei_hbm.at[1, pl.ds(base, n)], idx_d)

        def g_start(c, b):
            pltpu.async_copy(x_hbm.at[idx_s.at[c]], bufs[b], gsem[b])

        def g_wait(c, b):
            pltpu.make_async_copy(x_hbm.at[idx_s.at[c]], bufs[b], gsem[b]).wait()

        def s_start(c, b):
            pltpu.async_copy(bufs[b], agg_sh.at[idx_d.at[c]], ssem[b], add=True)

        def s_wait(c, b):
            pltpu.make_async_copy(bufs[b], agg_sh.at[idx_d.at[c]], ssem[b]).wait()

        for b in range(2):
            g_start(b, b)

        def body(g, _):
            for b in range(NB):
                c = g * NB + b
                g_wait(c, b)
                s_start(c, b)
                if b < 2:
                    @pl.when(g >= 1)
                    def _():
                        s_wait(c - 2, b - 2 + NB)
                else:
                    s_wait(c - 2, b - 2)
                if b < 2:
                    g_start(c + 2, b + 2)
                else:
                    @pl.when(g < n // NB - 1)
                    def _():
                        g_start(c + 2, b - 2)

            return 0

        lax.fori_loop(0, n // NB, body, 0)
        s_wait(n - 2, (n - 2) % NB)
        s_wait(n - 1, (n - 1) % NB)
        plsc.subcore_barrier()

        for r in range(rounds):
            u = r * NS + sid

            @pl.when(u < units)
            def _():
                pltpu.sync_copy(
                    agg_sh.at[pl.ds(u * K, K)],
                    agg_out.at[pl.ds(cid * N + u * K, K)],
                )

    return k


def _make_sc_deg(N, E, deg_w=16):
    n_chunk_rows = E // KD
    n = n_chunk_rows // (NC * NS)
    units = N // KD
    rounds = (units + NS - 1) // NS
    batch = 10
    assert n_chunk_rows * KD == E and n * NC * NS == n_chunk_rows
    assert units * KD == N and n % batch == 0

    mesh = plsc.VectorSubcoreMesh(core_axis_name="c", subcore_axis_name="s")

    @functools.partial(
        pl.kernel,
        mesh=mesh,
        out_type=jax.ShapeDtypeStruct((NC * N, deg_w), jnp.float32),
        scratch_types=[
            pltpu.VMEM((n, KD), jnp.int32),
            pltpu.VMEM((KD, deg_w), jnp.float32),
            pltpu.VMEM_SHARED((N, deg_w), jnp.float32),
            pltpu.SemaphoreType.DMA,
        ],
        compiler_params=pltpu.CompilerParams(use_tc_tiling_on_sc=False),
    )
    def k(ei_hbm, deg_out, idx_d, ones_v, deg_sh, sem):
        cid = lax.axis_index("c")
        sid = lax.axis_index("s")

        _fill(ones_v, KD, deg_w, 0.0)
        for r in range(rounds):
            u = r * NS + sid

            @pl.when(u < units)
            def _():
                pltpu.sync_copy(ones_v, deg_sh.at[pl.ds(u * KD, KD)])

        _fill(ones_v, KD, deg_w, 1.0)
        plsc.subcore_barrier()

        base = (cid * NS + sid) * n
        pltpu.sync_copy(ei_hbm.at[1, pl.ds(base, n)], idx_d)

        def body(g, _):
            for j in range(batch):
                c = g * batch + j
                pltpu.async_copy(ones_v, deg_sh.at[idx_d.at[c]], sem, add=True)
            for j in range(batch):
                c = g * batch + j
                pltpu.make_async_copy(ones_v, deg_sh.at[idx_d.at[c]], sem).wait()
            return 0

        lax.fori_loop(0, n // batch, body, 0)
        plsc.subcore_barrier()

        for r in range(rounds):
            u = r * NS + sid

            @pl.when(u < units)
            def _():
                pltpu.sync_copy(
                    deg_sh.at[pl.ds(u * KD, KD)],
                    deg_out.at[pl.ds(cid * N + u * KD, KD)],
                )

    return k


def _tc_layer(N, D, H, bn, deg_w, relu, w2_cols=None):
    grid = (N // bn,)

    def body(x_ref, agg_ref, agg2_ref, deg_ref, deg2_ref, ws_ref, wn_ref, b_ref,
             *rest):
        deg = (deg_ref[...] + deg2_ref[...])[:, :1]
        mean = (agg_ref[...] + agg2_ref[...]) / jnp.maximum(deg, 1.0)
        h = (
            jnp.dot(x_ref[...], ws_ref[...], preferred_element_type=jnp.float32)
            + jnp.dot(mean, wn_ref[...], preferred_element_type=jnp.float32)
            + b_ref[...]
        )
        if relu:
            h = jnp.maximum(h, 0.0)
        if w2_cols is not None:
            w2_ref, o_ref, y_ref = rest
            o_ref[...] = h
            y_ref[...] = jnp.dot(h, w2_ref[...], preferred_element_type=jnp.float32)
        else:
            (o_ref,) = rest
            o_ref[...] = h

    in_specs = [
        pl.BlockSpec((bn, D), lambda i: (i, 0)),
        pl.BlockSpec((bn, H), lambda i: (i, 0)),
        pl.BlockSpec((bn, H), lambda i: (i + N // bn, 0)),
        pl.BlockSpec((bn, deg_w), lambda i: (i, 0)),
        pl.BlockSpec((bn, deg_w), lambda i: (i + N // bn, 0)),
        pl.BlockSpec((D, H), lambda i: (0, 0)),
        pl.BlockSpec((H, H), lambda i: (0, 0)),
        pl.BlockSpec((1, H), lambda i: (0, 0)),
    ]
    out_shape = [jax.ShapeDtypeStruct((N, H), jnp.float32)]
    out_specs = [pl.BlockSpec((bn, H), lambda i: (i, 0))]
    if w2_cols is not None:
        in_specs.append(pl.BlockSpec((H, w2_cols), lambda i: (0, 0)))
        out_shape.append(jax.ShapeDtypeStruct((N, w2_cols), jnp.float32))
        out_specs.append(pl.BlockSpec((bn, w2_cols), lambda i: (i, 0)))

    return pl.pallas_call(
        body,
        grid=grid,
        in_specs=in_specs,
        out_specs=out_specs if w2_cols is not None else out_specs[0],
        out_shape=out_shape if w2_cols is not None else out_shape[0],
    )


def _tc_layer3(N, D, CP, C, bn, deg_w):
    grid = (N // bn,)

    def body(x_ref, agg_ref, agg2_ref, deg_ref, deg2_ref, ws_ref, b_ref, o_ref):
        deg = (deg_ref[...] + deg2_ref[...])[:, :1]
        mean = (agg_ref[...] + agg2_ref[...]) / jnp.maximum(deg, 1.0)
        h = (
            jnp.dot(x_ref[...], ws_ref[...], preferred_element_type=jnp.float32)
            + mean
            + b_ref[...]
        )
        col = lax.broadcasted_iota(jnp.int32, h.shape, 1)
        hm = jnp.where(col < C, h, -1e30)
        m = jnp.max(hm, axis=-1, keepdims=True)
        e = jnp.where(col < C, jnp.exp(hm - m), 0.0)
        s = jnp.sum(e, axis=-1, keepdims=True)
        o_ref[...] = (hm - m - jnp.log(s))[:, :C]

    return pl.pallas_call(
        body,
        grid=grid,
        in_specs=[
            pl.BlockSpec((bn, D), lambda i: (i, 0)),
            pl.BlockSpec((bn, CP), lambda i: (i, 0)),
            pl.BlockSpec((bn, CP), lambda i: (i + N // bn, 0)),
            pl.BlockSpec((bn, deg_w), lambda i: (i, 0)),
            pl.BlockSpec((bn, deg_w), lambda i: (i + N // bn, 0)),
            pl.BlockSpec((D, CP), lambda i: (0, 0)),
            pl.BlockSpec((1, CP), lambda i: (0, 0)),
        ],
        out_specs=pl.BlockSpec((bn, C), lambda i: (i, 0)),
        out_shape=jax.ShapeDtypeStruct((N, C), jnp.float32),
    )


def kernel(nfeat, edge_index, W_self1, W_neigh1, b1, W_self2, W_neigh2, b2,
           W_self3, W_neigh3, b3):
    N, D = nfeat.shape
    E = edge_index.shape[1]
    H = W_self1.shape[1]
    C = W_self3.shape[1]
    CP = 48
    deg_w = 16
    bn = 2000

    ei3 = edge_index.reshape(2, E // K, K)
    ei3_deg = edge_index.reshape(2, E // KD, KD)

    Wn3p = jnp.pad(W_neigh3, ((0, 0), (0, CP - C)))
    Ws3p = jnp.pad(W_self3, ((0, 0), (0, CP - C)))
    b3p = jnp.pad(b3, (0, CP - C)).reshape(1, CP)

    deg = _make_sc_deg(N, E, deg_w)(ei3_deg)
    agg1 = _make_sc_agg(N, E, D)(nfeat, ei3)
    h1 = _tc_layer(N, D, H, bn, deg_w, True)(
        nfeat, agg1, agg1, deg, deg, W_self1, W_neigh1, b1.reshape(1, H)
    )
    agg2 = _make_sc_agg(N, E, H)(h1, ei3)
    h2, y3 = _tc_layer(N, H, H, bn, deg_w, True, w2_cols=CP)(
        h1, agg2, agg2, deg, deg, W_self2, W_neigh2, b2.reshape(1, H), Wn3p
    )
    agg3 = _make_sc_agg(N, E, CP)(y3, ei3)
    return _tc_layer3(N, H, CP, C, bn, deg_w)(h2, agg3, agg3, deg, deg, Ws3p, b3p)


# speedup vs baseline: 1.6012x; 1.1764x over previous
import functools

import jax
import jax.numpy as jnp
from jax import lax
from jax.experimental import pallas as pl
from jax.experimental.pallas import tpu as pltpu
from jax.experimental.pallas import tpu_sc as plsc

NC = 2
NS = 16
K = 50
NB = 4
KD = 100


def _fill(ref, nrows, ncols, val):
    v = jnp.full((16,), val, jnp.float32)
    npieces = ncols // 16

    def body(k, _):
        i = k // npieces
        j = k % npieces
        ref[i, pl.ds(j * 16, 16)] = v
        return 0

    lax.fori_loop(0, nrows * npieces, body, 0)


def _make_sc_agg(N, E, D):
    n_chunk_rows = E // K
    n = n_chunk_rows // (NC * NS)
    units = N // K
    rounds = (units + NS - 1) // NS
    assert n_chunk_rows * K == E and n * NC * NS == n_chunk_rows
    assert units * K == N
    assert n % NB == 0 and n >= 2 * NB

    scratch = [
        pltpu.VMEM((n, K), jnp.int32),
        pltpu.VMEM((n, K), jnp.int32),
        pltpu.VMEM_SHARED((N, D), jnp.float32),
    ]
    scratch += [pltpu.VMEM((K, D), jnp.float32) for _ in range(NB)]
    scratch += [pltpu.SemaphoreType.DMA for _ in range(2 * NB)]

    mesh = plsc.VectorSubcoreMesh(core_axis_name="c", subcore_axis_name="s")

    @functools.partial(
        pl.kernel,
        mesh=mesh,
        out_type=jax.ShapeDtypeStruct((NC * N, D), jnp.float32),
        scratch_types=scratch,
        compiler_params=pltpu.CompilerParams(use_tc_tiling_on_sc=False),
    )
    def k(x_hbm, ei_hbm, agg_out, idx_s, idx_d, agg_sh, *bufsem):
        bufs = bufsem[:NB]
        gsem = bufsem[NB:2 * NB]
        ssem = bufsem[2 * NB:]
        cid = lax.axis_index("c")
        sid = lax.axis_index("s")

        _fill(bufs[0], K, D, 0.0)
        for r in range(rounds):
            u = r * NS + sid

            @pl.when(u < units)
            def _():
                pltpu.sync_copy(bufs[0], agg_sh.at[pl.ds(u * K, K)])

        plsc.subcore_barrier()

        base = (cid * NS + sid) * n
        pltpu.sync_copy(ei_hbm.at[0, pl.ds(base, n)], idx_s)
        pltpu.sync_copy(ei_hbm.at[1, pl.ds(base, n)], idx_d)

        def g_start(c, b):
            pltpu.async_copy(x_hbm.at[idx_s.at[c]], bufs[b], gsem[b])

        def g_wait(c, b):
            pltpu.make_async_copy(x_hbm.at[idx_s.at[c]], bufs[b], gsem[b]).wait()

        def s_start(c, b):
            pltpu.async_copy(bufs[b], agg_sh.at[idx_d.at[c]], ssem[b], add=True)

        def s_wait(c, b):
            pltpu.make_async_copy(bufs[b], agg_sh.at[idx_d.at[c]], ssem[b]).wait()

        for b in range(NB - 1):
            g_start(b, b)

        def body(g, _):
            for b in range(NB):
                c = g * NB + b
                g_wait(c, b)
                s_start(c, b)
                if b == 0:
                    @pl.when(g >= 1)
                    def _():
                        s_wait(c - 1, NB - 1)
                else:
                    s_wait(c - 1, b - 1)
                if b == 0:
                    g_start(c + NB - 1, NB - 1)
                else:
                    @pl.when(g < n // NB - 1)
                    def _():
                        g_start(c + NB - 1, b - 1)

            return 0

        lax.fori_loop(0, n // NB, body, 0)
        s_wait(n - 1, (n - 1) % NB)
        plsc.subcore_barrier()

        for r in range(rounds):
            u = r * NS + sid

            @pl.when(u < units)
            def _():
                pltpu.sync_copy(
                    agg_sh.at[pl.ds(u * K, K)],
                    agg_out.at[pl.ds(cid * N + u * K, K)],
                )

    return k


def _make_sc_deg(N, E, deg_w=16):
    n_chunk_rows = E // KD
    n = n_chunk_rows // (NC * NS)
    units = N // KD
    rounds = (units + NS - 1) // NS
    batch = 10
    assert n_chunk_rows * KD == E and n * NC * NS == n_chunk_rows
    assert units * KD == N and n % batch == 0

    mesh = plsc.VectorSubcoreMesh(core_axis_name="c", subcore_axis_name="s")

    @functools.partial(
        pl.kernel,
        mesh=mesh,
        out_type=jax.ShapeDtypeStruct((NC * N, deg_w), jnp.float32),
        scratch_types=[
            pltpu.VMEM((n, KD), jnp.int32),
            pltpu.VMEM((KD, deg_w), jnp.float32),
            pltpu.VMEM_SHARED((N, deg_w), jnp.float32),
            pltpu.SemaphoreType.DMA,
        ],
        compiler_params=pltpu.CompilerParams(use_tc_tiling_on_sc=False),
    )
    def k(ei_hbm, deg_out, idx_d, ones_v, deg_sh, sem):
        cid = lax.axis_index("c")
        sid = lax.axis_index("s")

        _fill(ones_v, KD, deg_w, 0.0)
        for r in range(rounds):
            u = r * NS + sid

            @pl.when(u < units)
            def _():
                pltpu.sync_copy(ones_v, deg_sh.at[pl.ds(u * KD, KD)])

        _fill(ones_v, KD, deg_w, 1.0)
        plsc.subcore_barrier()

        base = (cid * NS + sid) * n
        pltpu.sync_copy(ei_hbm.at[1, pl.ds(base, n)], idx_d)

        def body(g, _):
            for j in range(batch):
                c = g * batch + j
                pltpu.async_copy(ones_v, deg_sh.at[idx_d.at[c]], sem, add=True)
            for j in range(batch):
                c = g * batch + j
                pltpu.make_async_copy(ones_v, deg_sh.at[idx_d.at[c]], sem).wait()
            return 0

        lax.fori_loop(0, n // batch, body, 0)
        plsc.subcore_barrier()

        for r in range(rounds):
            u = r * NS + sid

            @pl.when(u < units)
            def _():
                pltpu.sync_copy(
                    deg_sh.at[pl.ds(u * KD, KD)],
                    deg_out.at[pl.ds(cid * N + u * KD, KD)],
                )

    return k


def _tc_layer(N, D, H, bn, deg_w, relu, w2_cols=None):
    grid = (N // bn,)

    def body(x_ref, agg_ref, agg2_ref, deg_ref, deg2_ref, ws_ref, wn_ref, b_ref,
             *rest):
        deg = (deg_ref[...] + deg2_ref[...])[:, :1]
        mean = (agg_ref[...] + agg2_ref[...]) / jnp.maximum(deg, 1.0)
        h = (
            jnp.dot(x_ref[...], ws_ref[...], preferred_element_type=jnp.float32)
            + jnp.dot(mean, wn_ref[...], preferred_element_type=jnp.float32)
            + b_ref[...]
        )
        if relu:
            h = jnp.maximum(h, 0.0)
        if w2_cols is not None:
            w2_ref, o_ref, y_ref = rest
            o_ref[...] = h
            y_ref[...] = jnp.dot(h, w2_ref[...], preferred_element_type=jnp.float32)
        else:
            (o_ref,) = rest
            o_ref[...] = h

    in_specs = [
        pl.BlockSpec((bn, D), lambda i: (i, 0)),
        pl.BlockSpec((bn, H), lambda i: (i, 0)),
        pl.BlockSpec((bn, H), lambda i: (i + N // bn, 0)),
        pl.BlockSpec((bn, deg_w), lambda i: (i, 0)),
        pl.BlockSpec((bn, deg_w), lambda i: (i + N // bn, 0)),
        pl.BlockSpec((D, H), lambda i: (0, 0)),
        pl.BlockSpec((H, H), lambda i: (0, 0)),
        pl.BlockSpec((1, H), lambda i: (0, 0)),
    ]
    out_shape = [jax.ShapeDtypeStruct((N, H), jnp.float32)]
    out_specs = [pl.BlockSpec((bn, H), lambda i: (i, 0))]
    if w2_cols is not None:
        in_specs.append(pl.BlockSpec((H, w2_cols), lambda i: (0, 0)))
        out_shape.append(jax.ShapeDtypeStruct((N, w2_cols), jnp.float32))
        out_specs.append(pl.BlockSpec((bn, w2_cols), lambda i: (i, 0)))

    return pl.pallas_call(
        body,
        grid=grid,
        in_specs=in_specs,
        out_specs=out_specs if w2_cols is not None else out_specs[0],
        out_shape=out_shape if w2_cols is not None else out_shape[0],
    )


def _tc_layer3(N, D, CP, C, bn, deg_w):
    grid = (N // bn,)

    def body(x_ref, agg_ref, agg2_ref, deg_ref, deg2_ref, ws_ref, b_ref, o_ref):
        deg = (deg_ref[...] + deg2_ref[...])[:, :1]
        mean = (agg_ref[...] + agg2_ref[...]) / jnp.maximum(deg, 1.0)
        h = (
            jnp.dot(x_ref[...], ws_ref[...], preferred_element_type=jnp.float32)
            + mean
            + b_ref[...]
        )
        col = lax.broadcasted_iota(jnp.int32, h.shape, 1)
        hm = jnp.where(col < C, h, -1e30)
        m = jnp.max(hm, axis=-1, keepdims=True)
        e = jnp.where(col < C, jnp.exp(hm - m), 0.0)
        s = jnp.sum(e, axis=-1, keepdims=True)
        o_ref[...] = (hm - m - jnp.log(s))[:, :C]

    return pl.pallas_call(
        body,
        grid=grid,
        in_specs=[
            pl.BlockSpec((bn, D), lambda i: (i, 0)),
            pl.BlockSpec((bn, CP), lambda i: (i, 0)),
            pl.BlockSpec((bn, CP), lambda i: (i + N // bn, 0)),
            pl.BlockSpec((bn, deg_w), lambda i: (i, 0)),
            pl.BlockSpec((bn, deg_w), lambda i: (i + N // bn, 0)),
            pl.BlockSpec((D, CP), lambda i: (0, 0)),
            pl.BlockSpec((1, CP), lambda i: (0, 0)),
        ],
        out_specs=pl.BlockSpec((bn, C), lambda i: (i, 0)),
        out_shape=jax.ShapeDtypeStruct((N, C), jnp.float32),
    )


def kernel(nfeat, edge_index, W_self1, W_neigh1, b1, W_self2, W_neigh2, b2,
           W_self3, W_neigh3, b3):
    N, D = nfeat.shape
    E = edge_index.shape[1]
    H = W_self1.shape[1]
    C = W_self3.shape[1]
    CP = 48
    deg_w = 16
    bn = 2000

    ei3 = edge_index.reshape(2, E // K, K)
    ei3_deg = edge_index.reshape(2, E // KD, KD)

    Wn3p = jnp.pad(W_neigh3, ((0, 0), (0, CP - C)))
    Ws3p = jnp.pad(W_self3, ((0, 0), (0, CP - C)))
    b3p = jnp.pad(b3, (0, CP - C)).reshape(1, CP)

    deg = _make_sc_deg(N, E, deg_w)(ei3_deg)
    agg1 = _make_sc_agg(N, E, D)(nfeat, ei3)
    h1 = _tc_layer(N, D, H, bn, deg_w, True)(
        nfeat, agg1, agg1, deg, deg, W_self1, W_neigh1, b1.reshape(1, H)
    )
    agg2 = _make_sc_agg(N, E, H)(h1, ei3)
    h2, y3 = _tc_layer(N, H, H, bn, deg_w, True, w2_cols=CP)(
        h1, agg2, agg2, deg, deg, W_self2, W_neigh2, b2.reshape(1, H), Wn3p
    )
    agg3 = _make_sc_agg(N, E, CP)(y3, ei3)
    return _tc_layer3(N, H, CP, C, bn, deg_w)(h2, agg3, agg3, deg, deg, Ws3p, b3p)
